# baseline (speedup 1.0000x reference)
import functools

import jax
import jax.numpy as jnp
from jax import lax
from jax.experimental import pallas as pl
from jax.experimental.pallas import tpu as pltpu
from jax.experimental.pallas import tpu_sc as plsc

N_EXPERTS = 64
TOP_K = 8
N_GROUP = 8
TOPK_GROUP = 4
SCALE = 2.5
PER_GROUP = N_EXPERTS // N_GROUP

TOKENS = 16384
CHUNKS = 4
CT = TOKENS // CHUNKS
BT = 2048
NW = 32
TPB = CT // NW
L = 16
NCHUNK = TPB // L
WPB = BT // TPB


def _scores_block(x_ref, w_ref, s_out_ref):
    logits = jax.lax.dot_general(
        w_ref[...], x_ref[...],
        dimension_numbers=(((1,), (1,)), ((), ())),
        preferred_element_type=jnp.float32,
    )
    sig = jax.nn.sigmoid(logits)
    for w in range(WPB):
        s_out_ref[w] = sig[:, w * TPB : (w + 1) * TPB]


def _tc_scores(x_chunk, weight):
    ct, d = x_chunk.shape
    return pl.pallas_call(
        _scores_block,
        grid=(ct // BT,),
        in_specs=[
            pl.BlockSpec((BT, d), lambda i: (i, 0)),
            pl.BlockSpec((N_EXPERTS, d), lambda i: (0, 0)),
        ],
        out_specs=pl.BlockSpec((WPB, N_EXPERTS, TPB), lambda i: (i, 0, 0)),
        out_shape=jax.ShapeDtypeStruct((NW, N_EXPERTS, TPB), jnp.float32),
    )(x_chunk, weight)


def _tree_max(vs):
    while len(vs) > 1:
        vs = [jnp.maximum(vs[i], vs[i + 1]) for i in range(0, len(vs) - 1, 2)] + (
            [vs[-1]] if len(vs) % 2 else []
        )
    return vs[0]


def _tree_min(vs):
    while len(vs) > 1:
        vs = [jnp.minimum(vs[i], vs[i + 1]) for i in range(0, len(vs) - 1, 2)] + (
            [vs[-1]] if len(vs) % 2 else []
        )
    return vs[0]


def _sc_route_body(scores_hbm, biasb_hbm, wout_hbm, iout_hbm,
                   sc_v, bias_v, work_v, wout_v, iout_v):
    neg_inf = jnp.float32(-jnp.inf)
    wid = lax.axis_index("s") * 2 + lax.axis_index("c")
    base = wid * (N_EXPERTS * TPB)
    pltpu.sync_copy(scores_hbm.at[pl.ds(base, N_EXPERTS * TPB)], sc_v)
    pltpu.sync_copy(biasb_hbm, bias_v)

    def chunk(t, carry):
        col = t * L
        lanes = lax.iota(jnp.int32, L) + col

        gm = []
        for g in range(N_GROUP):
            s8 = [
                sc_v[pl.ds((g * PER_GROUP + j) * TPB + col, L)]
                + bias_v[pl.ds((g * PER_GROUP + j) * L, L)]
                for j in range(PER_GROUP)
            ]
            m1 = _tree_max(s8)
            am1 = _tree_min(
                [jnp.where(s8[j] == m1, jnp.int32(j), jnp.int32(PER_GROUP))
                 for j in range(PER_GROUP)]
            )
            m2 = _tree_max(
                [jnp.where(am1 == j, neg_inf, s8[j]) for j in range(PER_GROUP)]
            )
            gm.append(m1 + m2)
            for j in range(PER_GROUP):
                work_v[pl.ds((g * PER_GROUP + j) * TPB + col, L)] = s8[j]

        gsel = [jnp.zeros((L,), jnp.bool_) for _ in range(N_GROUP)]
        for _ in range(TOPK_GROUP):
            m = _tree_max(gm)
            amg = _tree_min(
                [jnp.where(gm[g] == m, jnp.int32(g), jnp.int32(N_GROUP))
                 for g in range(N_GROUP)]
            )
            for g in range(N_GROUP):
                hit = amg == g
                gsel[g] = gsel[g] | hit
                gm[g] = jnp.where(hit, neg_inf, gm[g])

        for g in range(N_GROUP):
            for j in range(PER_GROUP):
                off = (g * PER_GROUP + j) * TPB + col
                v = work_v[pl.ds(off, L)]
                work_v[pl.ds(off, L)] = jnp.where(gsel[g], v, 0.0)

        wvs = []
        for k in range(TOP_K):
            v = [work_v[pl.ds(e * TPB + col, L)] for e in range(N_EXPERTS)]
            m = _tree_max(v)
            am = _tree_min(
                [jnp.where(v[e] == m, jnp.int32(e), jnp.int32(N_EXPERTS))
                 for e in range(N_EXPERTS)]
            )
            wv = plsc.load_gather(sc_v, [am * TPB + lanes])
            wvs.append(wv)
            iout_v[pl.ds(k * TPB + col, L)] = am
            plsc.store_scatter(work_v, [am * TPB + lanes], jnp.full((L,), neg_inf))

        denom = wvs[0]
        for wv in wvs[1:]:
            denom = denom + wv
        inv = SCALE / (denom + 1e-20)
        for k in range(TOP_K):
            wout_v[pl.ds(k * TPB + col, L)] = wvs[k] * inv
        return carry

    lax.fori_loop(0, NCHUNK, chunk, 0)

    obase = wid * (TOP_K * TPB)
    pltpu.sync_copy(wout_v, wout_hbm.at[pl.ds(obase, TOP_K * TPB)])
    pltpu.sync_copy(iout_v, iout_hbm.at[pl.ds(obase, TOP_K * TPB)])


def _sc_route(scores_flat, bias_flat):
    mesh = plsc.VectorSubcoreMesh(core_axis_name="c", subcore_axis_name="s")
    return pl.kernel(
        _sc_route_body,
        mesh=mesh,
        out_type=[
            jax.ShapeDtypeStruct((NW * TOP_K * TPB,), jnp.float32),
            jax.ShapeDtypeStruct((NW * TOP_K * TPB,), jnp.int32),
        ],
        scratch_types=[
            pltpu.VMEM((N_EXPERTS * TPB,), jnp.float32),
            pltpu.VMEM((N_EXPERTS * L,), jnp.float32),
            pltpu.VMEM((N_EXPERTS * TPB,), jnp.float32),
            pltpu.VMEM((TOP_K * TPB,), jnp.float32),
            pltpu.VMEM((TOP_K * TPB,), jnp.int32),
        ],
        compiler_params=pltpu.CompilerParams(needs_layout_passes=False),
    )(scores_flat, bias_flat)


@jax.jit
def _gate(hidden_states, weight, bias_flat):
    w_parts = []
    i_parts = []
    for c in range(CHUNKS):
        x_c = lax.slice_in_dim(hidden_states, c * CT, (c + 1) * CT, axis=0)
        scores_c = _tc_scores(x_c, weight)
        w_flat, i_flat = _sc_route(scores_c.reshape(-1), bias_flat)
        w_parts.append(
            w_flat.reshape(NW, TOP_K, TPB).transpose(0, 2, 1).reshape(CT, TOP_K)
        )
        i_parts.append(
            i_flat.reshape(NW, TOP_K, TPB).transpose(0, 2, 1).reshape(CT, TOP_K)
        )
    return (
        jnp.concatenate(w_parts, axis=0),
        jnp.concatenate(i_parts, axis=0),
    )


def kernel(hidden_states, weight, e_score_correction_bias):
    bias_flat = jnp.broadcast_to(
        e_score_correction_bias.reshape(N_EXPERTS, 1), (N_EXPERTS, L)
    ).reshape(-1)
    return _gate(hidden_states, weight, bias_flat)


# speedup vs baseline: 3.6724x; 3.6724x over previous
import functools

import jax
import jax.numpy as jnp
from jax.experimental import pallas as pl
from jax.experimental.pallas import tpu as pltpu

N_EXPERTS = 64
TOP_K = 8
N_GROUP = 8
TOPK_GROUP = 4
SCALE = 2.5
PER_GROUP = N_EXPERTS // N_GROUP

BT = 2048


def _gate_block(x_ref, w_ref, b_ref, w_out_ref, i_out_ref):
    bt = x_ref.shape[0]
    neg_inf = jnp.float32(-jnp.inf)

    logits = jax.lax.dot_general(
        w_ref[...], x_ref[...],
        dimension_numbers=(((1,), (1,)), ((), ())),
        preferred_element_type=jnp.float32,
    )
    scores = jax.nn.sigmoid(logits)
    sfc = scores + b_ref[...]

    iota_e = jax.lax.broadcasted_iota(jnp.int32, (N_EXPERTS, bt), 0)
    iota_g8 = jax.lax.broadcasted_iota(jnp.int32, (PER_GROUP, bt), 0)

    gs_rows = []
    for g in range(N_GROUP):
        sub = sfc[g * PER_GROUP : (g + 1) * PER_GROUP, :]
        m1 = jnp.max(sub, axis=0, keepdims=True)
        am1 = jnp.min(jnp.where(sub == m1, iota_g8, PER_GROUP), axis=0, keepdims=True)
        m2 = jnp.max(jnp.where(iota_g8 == am1, neg_inf, sub), axis=0, keepdims=True)
        gs_rows.append(m1 + m2)
    gs8 = jnp.concatenate(gs_rows, axis=0)

    iota_grp = jax.lax.broadcasted_iota(jnp.int32, (N_GROUP, bt), 0)
    gmask8 = jnp.zeros((N_GROUP, bt), jnp.bool_)
    work_g = gs8
    for _ in range(TOPK_GROUP):
        m = jnp.max(work_g, axis=0, keepdims=True)
        amg = jnp.min(jnp.where(work_g == m, iota_grp, N_GROUP), axis=0, keepdims=True)
        sel = iota_grp == amg
        gmask8 = gmask8 | sel
        work_g = jnp.where(sel, neg_inf, work_g)

    tmp = jnp.concatenate(
        [
            jnp.where(
                gmask8[g : g + 1, :],
                sfc[g * PER_GROUP : (g + 1) * PER_GROUP, :],
                0.0,
            )
            for g in range(N_GROUP)
        ],
        axis=0,
    )

    vals = []
    idxs = []
    work = tmp
    for k in range(TOP_K):
        m = jnp.max(work, axis=0, keepdims=True)
        am = jnp.min(jnp.where(work == m, iota_e, N_EXPERTS), axis=0, keepdims=True)
        hit = iota_e == am
        wv = jnp.max(jnp.where(hit, scores, neg_inf), axis=0, keepdims=True)
        vals.append(wv)
        idxs.append(am)
        work = jnp.where(hit, neg_inf, work)

    denom = vals[0]
    for v in vals[1:]:
        denom = denom + v
    denom = denom + 1e-20
    w_out_ref[...] = jnp.concatenate(vals, axis=0) / denom * SCALE
    i_out_ref[...] = jnp.concatenate(idxs, axis=0)


@jax.jit
def _gate(hidden_states, weight, bias2d):
    s, d = hidden_states.shape
    grid = (s // BT,)
    w_t, i_t = pl.pallas_call(
        _gate_block,
        grid=grid,
        in_specs=[
            pl.BlockSpec((BT, d), lambda i: (i, 0)),
            pl.BlockSpec((N_EXPERTS, d), lambda i: (0, 0)),
            pl.BlockSpec((N_EXPERTS, 1), lambda i: (0, 0)),
        ],
        out_specs=[
            pl.BlockSpec((TOP_K, BT), lambda i: (0, i)),
            pl.BlockSpec((TOP_K, BT), lambda i: (0, i)),
        ],
        out_shape=[
            jax.ShapeDtypeStruct((TOP_K, s), jnp.float32),
            jax.ShapeDtypeStruct((TOP_K, s), jnp.int32),
        ],
    )(hidden_states, weight, bias2d)
    return w_t.T, i_t.T


def kernel(hidden_states, weight, e_score_correction_bias):
    bias2d = e_score_correction_bias.reshape(N_EXPERTS, 1)
    topk_weight, topk_idx = _gate(hidden_states, weight, bias2d)
    return (topk_weight, topk_idx)
